# baseline (speedup 1.0000x reference)
import functools

import jax
import jax.numpy as jnp
from jax import lax
from jax.experimental import pallas as pl
from jax.experimental.pallas import tpu as pltpu
from jax.experimental.pallas import tpu_sc as plsc

N = 9
C = 160
CV = C // 16
E = 81
EP = 96
IBLK = 2 * EP + 16

OFF_X = 0
OFF_W = OFF_X + N * C
OFF_GWT = OFF_W + N * C
OFF_GB = OFF_GWT + N * 16
OFF_FCW = OFF_GB + 16
OFF_GAM = OFF_FCW + 16
OFF_BET = OFF_GAM + 16
OFF_FCB = OFF_BET + 16
FBLK = OFF_FCB + 16

_PAIRS = []
for _p in range(N):
    for _q in range(N):
        _dy, _dx = _q // 3 - _p // 3, _q % 3 - _p % 3
        if abs(_dy) <= 1 and abs(_dx) <= 1:
            _PAIRS.append((_p, _q, (_dy + 1) * 3 + (_dx + 1)))


def _body(fin_hbm, ein_hbm, o1_hbm, o2_hbm, fblk, iblk, fv, av, red, tblk,
          st1, st2):
    wid = lax.axis_index("s") * 2 + lax.axis_index("c")
    zf = jnp.zeros((16,), jnp.float32)

    @pl.when(wid == 0)
    def _():
        pltpu.sync_copy(fin_hbm, fblk)
        pltpu.sync_copy(ein_hbm, iblk)
        lane = lax.broadcasted_iota(jnp.int32, (16,), 0)
        zid = iblk[pl.ds(2 * EP, 16)]

        def lanesum(v):
            red[...] = zf
            plsc.addupdate_scatter(red, [zid], v)
            return plsc.load_gather(red, [zid])

        acc = [jnp.zeros((16,), jnp.float32) for _ in range(N)]
        for v in range(CV):
            xc = [fblk[pl.ds(OFF_X + q * C + v * 16, 16)] for q in range(N)]
            wc = [fblk[pl.ds(OFF_W + t * C + v * 16, 16)] for t in range(N)]
            for p, q, t in _PAIRS:
                acc[p] = acc[p] + xc[q] * wc[t]
        for i in range(16):
            tblk[pl.ds(i * 16, 16)] = zf
        for p in range(N):
            tblk[pl.ds(p * 16, 16)] = acc[p]
        h = plsc.load_gather(tblk, [lane * 16])
        for l in range(1, 16):
            h = h + plsc.load_gather(tblk, [lane * 16 + l])

        mean = lanesum(h) * (1.0 / N)
        d = jnp.where(lane < N, h - mean, 0.0)
        var = lanesum(d * d) * (1.0 / N)
        vv = var + 1e-5
        y = plsc.bitcast(0x5F3759DF - (plsc.bitcast(vv, jnp.int32) >> 1),
                         jnp.float32)
        for _ in range(3):
            y = y * (1.5 - 0.5 * vv * y * y)
        gam = fblk[pl.ds(OFF_GAM, 16)]
        bet = fblk[pl.ds(OFF_BET, 16)]
        feats = jnp.where(lane < N, jnp.maximum(d * y * gam + bet, 0.0), 0.0)

        fv[...] = feats
        av[...] = jnp.zeros((16,), jnp.float32)
        for k in range(EP // 16):
            sidx = iblk[pl.ds(k * 16, 16)]
            didx = iblk[pl.ds(EP + k * 16, 16)]
            msgs = plsc.load_gather(fv, [sidx])
            rem = E - k * 16
            if rem >= 16:
                plsc.addupdate_scatter(av, [didx], msgs)
            else:
                plsc.addupdate_scatter(av, [didx], msgs, mask=lane < rem)

        h2 = fblk[pl.ds(OFF_GB, 16)]
        for j in range(N):
            bj = plsc.load_gather(av, [zid + j] if j else [zid])
            h2 = h2 + fblk[pl.ds(OFF_GWT + j * 16, 16)] * bj
        h2 = jnp.maximum(h2, 0.0)

        r = lanesum(fblk[pl.ds(OFF_FCW, 16)] * h2)
        r = r + fblk[pl.ds(OFF_FCB, 16)]
        tz = 1.0 - 2.0 / (jnp.exp(2.0 * r) + 1.0)

        st1[...] = h2
        st2[...] = tz
        pltpu.sync_copy(st1, o1_hbm)
        pltpu.sync_copy(st2, o2_hbm)


@functools.partial(
    pl.kernel,
    out_type=(jax.ShapeDtypeStruct((16,), jnp.float32),
              jax.ShapeDtypeStruct((16,), jnp.float32)),
    mesh=plsc.VectorSubcoreMesh(core_axis_name="c", subcore_axis_name="s",
                                num_cores=2, num_subcores=16),
    compiler_params=pltpu.CompilerParams(needs_layout_passes=False),
    scratch_types=[
        pltpu.VMEM((FBLK,), jnp.float32),
        pltpu.VMEM((IBLK,), jnp.int32),
        pltpu.VMEM((16,), jnp.float32),
        pltpu.VMEM((16,), jnp.float32),
        pltpu.VMEM((16,), jnp.float32),
        pltpu.VMEM((256,), jnp.float32),
        pltpu.VMEM((16,), jnp.float32),
        pltpu.VMEM((16,), jnp.float32),
    ],
)
def _dynamics_sc(fin_hbm, ein_hbm, o1_hbm, o2_hbm, fblk, iblk, fv, av, red,
                 tblk, st1, st2):
    _body(fin_hbm, ein_hbm, o1_hbm, o2_hbm, fblk, iblk, fv, av, red, tblk,
          st1, st2)


def kernel(state, action, conv_w, bn_gamma, bn_beta, gcn_w, gcn_b, fc_w, fc_b,
           edge_index):
    x2d = jnp.concatenate([state.reshape(-1, N), action.reshape(-1, N)], axis=0)
    xsp = x2d.T.reshape(-1)
    wsp = conv_w.reshape(C, N).T.reshape(-1)
    gwt = jnp.pad(gcn_w.T, ((0, 0), (0, 16 - N))).reshape(-1)
    gbp = jnp.pad(gcn_b, (0, 16 - N))
    fcwp = jnp.pad(fc_w.reshape(-1), (0, 16 - N))
    gam = jnp.broadcast_to(bn_gamma.reshape(()), (16,))
    bet = jnp.broadcast_to(bn_beta.reshape(()), (16,))
    fcb = jnp.broadcast_to(fc_b.reshape(()), (16,))
    fin = jnp.concatenate([xsp, wsp, gwt, gbp, fcwp, gam, bet, fcb])
    ein = jnp.pad(jnp.pad(edge_index, ((0, 0), (0, EP - E))).reshape(-1),
                  (0, 16))

    o1, o2 = _dynamics_sc(fin, ein.astype(jnp.int32))
    return (o1[:N].reshape(1, 1, 3, 3), o2[:1])
